# Initial kernel scaffold; baseline (speedup 1.0000x reference)
import jax
import jax.numpy as jnp
from jax.experimental import pallas as pl


def kernel(X, W_rel, W_0, update_gate_W, update_gate_U, W_glob, b_glob, x_indices, edge_index):
    raise NotImplementedError("write your pallas kernel here")


# speedup vs baseline: 51.7533x; 51.7533x over previous
import functools

import jax
import jax.numpy as jnp
from jax import lax
from jax.experimental import pallas as pl
from jax.experimental.pallas import tpu as pltpu
from jax.experimental.pallas import tpu_sc as plsc

_N = 32
_D = 128
_R = 3
_S = 32
_E = 64
_V = 40000
_VB = 2048
_VPAD = 40960
_NB = _VPAD // _VB
_NEG = -1e30


def _sc_gather(table, idx_flat):
    n_rows = idx_flat.shape[0]
    nw = 32
    bpw = n_rows // nw
    mesh = plsc.VectorSubcoreMesh(core_axis_name="c", subcore_axis_name="s")

    @functools.partial(
        pl.kernel,
        mesh=mesh,
        out_type=jax.ShapeDtypeStruct((n_rows, _D), jnp.float32),
        scratch_types=[
            pltpu.VMEM((bpw,), jnp.int32),
            pltpu.VMEM((bpw, _D), jnp.float32),
            pltpu.SemaphoreType.DMA,
        ],
    )
    def gather_kernel(table_hbm, idx_hbm, out_hbm, idx_v, rows_v, sem):
        wid = lax.axis_index("s") * 2 + lax.axis_index("c")
        base = wid * bpw
        pltpu.sync_copy(idx_hbm.at[pl.ds(base, bpw)], idx_v)
        pltpu.async_copy(table_hbm.at[idx_v], rows_v, sem).wait()
        pltpu.sync_copy(rows_v, out_hbm.at[pl.ds(base, bpw)])

    return gather_kernel(table, idx_flat)


def _stage_body(src_ref, dst_ref, g3_ref, gflat_ref, wrel_ref, w0_ref,
                ugw_ref, ugu_ref, x1_ref, ng_ref, prop_ref):
    src = src_ref[...]
    dst = dst_ref[...]
    nodes = lax.broadcasted_iota(jnp.int32, (_S * _R, _N, _E), 1)
    eq_d = (dst[:, None, :] == nodes).astype(jnp.float32)
    deg = 1.0 + jnp.sum(eq_d, axis=2)
    dinv = lax.rsqrt(deg)
    mask0 = (dst == 0).astype(jnp.float32)[:, None, :]
    eq_s = (src[:, None, :] == nodes).astype(jnp.float32)
    cnt0 = jnp.sum(mask0 * eq_s, axis=2)
    dinv0 = dinv[:, 0:1]
    node2 = lax.broadcasted_iota(jnp.int32, (_S * _R, _N), 1)
    self0 = (node2 == 0).astype(jnp.float32)
    c = dinv0 * dinv * cnt0 + self0 * dinv0 * dinv0
    c3 = c.reshape(_S, _R, _N)

    g3 = g3_ref[...]
    y = lax.dot_general(c3, g3, (((2,), (1,)), ((0,), (0,))),
                        preferred_element_type=jnp.float32)
    wrel = wrel_ref[...]
    prop = jnp.dot(g3[:, 0, :], w0_ref[...],
                   preferred_element_type=jnp.float32)
    for r in range(_R):
        prop = prop + jnp.dot(y[:, r, :], wrel[r],
                              preferred_element_type=jnp.float32)
    ng_ref[...] = jnp.dot(gflat_ref[...], ugw_ref[...],
                          preferred_element_type=jnp.float32)
    prop_ref[...] = prop
    ugu = ugu_ref[...]

    def body(t, mem0):
        pg = jnp.dot(mem0, ugu, preferred_element_type=jnp.float32)
        ngt = ng_ref[pl.ds(t, 1), :]
        pt = prop_ref[pl.ds(t, 1), :]
        gate = jax.nn.sigmoid(ngt + pg)
        new = gate * pt + (1.0 - gate) * mem0
        x1_ref[pl.ds(t, 1), :] = jnp.where(new >= 0.0, new, 0.01 * new)
        return new

    lax.fori_loop(0, _S, body, jnp.zeros((1, _D), jnp.float32))


def _logits_body(x1_ref, w_ref, b_ref, out_ref):
    j = pl.program_id(0)
    logits = jnp.dot(x1_ref[...], w_ref[...],
                     preferred_element_type=jnp.float32) + b_ref[...]
    off = pl.multiple_of(j * _VB, _VB)
    out_ref[:, pl.ds(off, _VB)] = logits

    @pl.when(j == _NB - 1)
    def _():
        full = out_ref[...]
        m = jnp.max(full, axis=1, keepdims=True)
        s = jnp.sum(jnp.exp(full - m), axis=1, keepdims=True)
        out_ref[...] = full - m - jnp.log(s)


def kernel(X, W_rel, W_0, update_gate_W, update_gate_U, W_glob, b_glob,
           x_indices, edge_index):
    idx_flat = x_indices.reshape(-1).astype(jnp.int32)
    g = _sc_gather(X, idx_flat)
    g3 = g.reshape(_S, _N, _D)
    gflat = g.reshape(_S, _N * _D)

    ei = edge_index.reshape(_S * _R, 2, _E).astype(jnp.int32)
    src = ei[:, 0, :]
    dst = ei[:, 1, :]

    x1 = pl.pallas_call(
        _stage_body,
        out_shape=jax.ShapeDtypeStruct((_S, _D), jnp.float32),
        scratch_shapes=[pltpu.VMEM((_S, _D), jnp.float32),
                        pltpu.VMEM((_S, _D), jnp.float32)],
    )(src, dst, g3, gflat, W_rel, W_0, update_gate_W, update_gate_U)

    w_pad = jnp.pad(W_glob, ((0, 0), (0, _VPAD - _V)))
    b_pad = jnp.pad(b_glob, (0, _VPAD - _V),
                    constant_values=_NEG).reshape(1, _VPAD)

    logp = pl.pallas_call(
        _logits_body,
        grid=(_NB,),
        in_specs=[
            pl.BlockSpec((_S, _D), lambda j: (0, 0)),
            pl.BlockSpec((_D, _VB), lambda j: (0, j)),
            pl.BlockSpec((1, _VB), lambda j: (0, j)),
        ],
        out_specs=pl.BlockSpec((_S, _VPAD), lambda j: (0, 0)),
        out_shape=jax.ShapeDtypeStruct((_S, _VPAD), jnp.float32),
    )(x1, w_pad, b_pad)

    preds_globals = logp[:, :_V]
    preds_senses = jnp.zeros((_S,), dtype=jnp.float32)
    return (preds_globals, preds_senses)


# speedup vs baseline: 58.2251x; 1.1251x over previous
import functools

import jax
import jax.numpy as jnp
from jax import lax
from jax.experimental import pallas as pl
from jax.experimental.pallas import tpu as pltpu
from jax.experimental.pallas import tpu_sc as plsc

_N = 32
_D = 128
_R = 3
_S = 32
_E = 64
_V = 40000
_VB = 2048
_VPAD = 40960
_NB = _VPAD // _VB
_NEG = -1e30


def _sc_gather(table, idx_flat):
    n_rows = idx_flat.shape[0]
    nw = 32
    bpw = n_rows // nw
    mesh = plsc.VectorSubcoreMesh(core_axis_name="c", subcore_axis_name="s")

    @functools.partial(
        pl.kernel,
        mesh=mesh,
        out_type=jax.ShapeDtypeStruct((n_rows, _D), jnp.float32),
        scratch_types=[
            pltpu.VMEM((bpw,), jnp.int32),
            pltpu.VMEM((bpw, _D), jnp.float32),
            pltpu.SemaphoreType.DMA,
        ],
    )
    def gather_kernel(table_hbm, idx_hbm, out_hbm, idx_v, rows_v, sem):
        wid = lax.axis_index("s") * 2 + lax.axis_index("c")
        base = wid * bpw
        pltpu.sync_copy(idx_hbm.at[pl.ds(base, bpw)], idx_v)
        pltpu.async_copy(table_hbm.at[idx_v], rows_v, sem).wait()
        pltpu.sync_copy(rows_v, out_hbm.at[pl.ds(base, bpw)])

    return gather_kernel(table, idx_flat)


def _stage_body(src_ref, dst_ref, g3_ref, gflat_ref, wrel_ref, w0_ref,
                ugw_ref, ugu_ref, x1_ref, ng_ref, prop_ref):
    src = src_ref[...]
    dst = dst_ref[...]
    nodes = lax.broadcasted_iota(jnp.int32, (_S * _R, _N, _E), 1)
    eq_d = (dst[:, None, :] == nodes).astype(jnp.float32)
    deg = 1.0 + jnp.sum(eq_d, axis=2)
    dinv = lax.rsqrt(deg)
    mask0 = (dst == 0).astype(jnp.float32)[:, None, :]
    eq_s = (src[:, None, :] == nodes).astype(jnp.float32)
    cnt0 = jnp.sum(mask0 * eq_s, axis=2)
    dinv0 = dinv[:, 0:1]
    node2 = lax.broadcasted_iota(jnp.int32, (_S * _R, _N), 1)
    self0 = (node2 == 0).astype(jnp.float32)
    c = dinv0 * dinv * cnt0 + self0 * dinv0 * dinv0
    c3 = c.reshape(_S, _R, _N)

    g3 = g3_ref[...]
    y = lax.dot_general(c3, g3, (((2,), (1,)), ((0,), (0,))),
                        preferred_element_type=jnp.float32)
    wrel = wrel_ref[...]
    prop = jnp.dot(g3[:, 0, :], w0_ref[...],
                   preferred_element_type=jnp.float32)
    for r in range(_R):
        prop = prop + jnp.dot(y[:, r, :], wrel[r],
                              preferred_element_type=jnp.float32)
    ng_ref[...] = jnp.dot(gflat_ref[...], ugw_ref[...],
                          preferred_element_type=jnp.float32)
    prop_ref[...] = prop
    ugu = ugu_ref[...]

    def body(t, mem0):
        pg = jnp.dot(mem0, ugu, preferred_element_type=jnp.float32)
        ngt = ng_ref[pl.ds(t, 1), :]
        pt = prop_ref[pl.ds(t, 1), :]
        gate = jax.nn.sigmoid(ngt + pg)
        new = gate * pt + (1.0 - gate) * mem0
        x1_ref[pl.ds(t, 1), :] = jnp.where(new >= 0.0, new, 0.01 * new)
        return new

    lax.fori_loop(0, _S, body, jnp.zeros((1, _D), jnp.float32))


def _logits_body(x1_ref, w_ref, b_ref, out_ref, buf_ref, m_ref, s_ref):
    p = pl.program_id(0)
    j = pl.program_id(1)

    @pl.when(p == 0)
    def _():
        logits = jnp.dot(x1_ref[...], w_ref[...],
                         preferred_element_type=jnp.float32) + b_ref[...]
        col = j * _VB + lax.broadcasted_iota(jnp.int32, (_S, _VB), 1)
        logits = jnp.where(col < _V, logits, _NEG)
        buf_ref[j] = logits

        @pl.when(j == 0)
        def _():
            m_ref[...] = jnp.full((_S, 1), _NEG, jnp.float32)
            s_ref[...] = jnp.zeros((_S, 1), jnp.float32)

        m_old = m_ref[...]
        m_new = jnp.maximum(m_old, jnp.max(logits, axis=1, keepdims=True))
        s_ref[...] = (s_ref[...] * jnp.exp(m_old - m_new)
                      + jnp.sum(jnp.exp(logits - m_new), axis=1,
                                keepdims=True))
        m_ref[...] = m_new

    @pl.when(p == 1)
    def _():
        lse = m_ref[...] + jnp.log(s_ref[...])
        out_ref[...] = buf_ref[j] - lse


def kernel(X, W_rel, W_0, update_gate_W, update_gate_U, W_glob, b_glob,
           x_indices, edge_index):
    idx_flat = x_indices.reshape(-1).astype(jnp.int32)
    g = _sc_gather(X, idx_flat)
    g3 = g.reshape(_S, _N, _D)
    gflat = g.reshape(_S, _N * _D)

    ei = edge_index.reshape(_S * _R, 2, _E).astype(jnp.int32)
    src = ei[:, 0, :]
    dst = ei[:, 1, :]

    x1 = pl.pallas_call(
        _stage_body,
        out_shape=jax.ShapeDtypeStruct((_S, _D), jnp.float32),
        scratch_shapes=[pltpu.VMEM((_S, _D), jnp.float32),
                        pltpu.VMEM((_S, _D), jnp.float32)],
    )(src, dst, g3, gflat, W_rel, W_0, update_gate_W, update_gate_U)

    preds_globals = pl.pallas_call(
        _logits_body,
        grid=(2, _NB),
        in_specs=[
            pl.BlockSpec((_S, _D), lambda p, j: (0, 0)),
            pl.BlockSpec((_D, _VB), lambda p, j: (0, j * (1 - p))),
            pl.BlockSpec((1, _VB), lambda p, j: (0, j * (1 - p))),
        ],
        out_specs=pl.BlockSpec((_S, _VB), lambda p, j: (0, p * j)),
        out_shape=jax.ShapeDtypeStruct((_S, _V), jnp.float32),
        scratch_shapes=[pltpu.VMEM((_NB, _S, _VB), jnp.float32),
                        pltpu.VMEM((_S, 1), jnp.float32),
                        pltpu.VMEM((_S, 1), jnp.float32)],
    )(x1, W_glob, b_glob.reshape(1, _V))

    preds_senses = jnp.zeros((_S,), dtype=jnp.float32)
    return (preds_globals, preds_senses)
